# baseline (speedup 1.0000x reference)
import functools

import jax
import jax.numpy as jnp
from jax import lax
from jax.experimental import pallas as pl
from jax.experimental.pallas import tpu as pltpu
from jax.experimental.pallas import tpu_sc as plsc

_NC, _NS, _L = 2, 16, 16
_NW = _NC * _NS


def _sc_gather_add(x_nc, idx_n, table):
    N, C = x_nc.shape
    n_per_w = N // _NW
    CH = 16
    n_ch = n_per_w // CH
    mesh = plsc.VectorSubcoreMesh(core_axis_name="c", subcore_axis_name="s")

    @functools.partial(
        pl.kernel,
        out_type=jax.ShapeDtypeStruct((N, C), jnp.float32),
        mesh=mesh,
        scratch_types=[
            pltpu.VMEM((n_per_w,), jnp.int32),
            pltpu.VMEM((CH, C), jnp.float32),
            pltpu.VMEM((CH, C), jnp.float32),
            pltpu.SemaphoreType.DMA,
            pltpu.SemaphoreType.DMA,
            pltpu.SemaphoreType.DMA,
        ],
    )
    def k(x_hbm, idx_hbm, tab_hbm, out_hbm, idx_v, x_v, r_v, sx, sr, so):
        wid = lax.axis_index("s") * _NC + lax.axis_index("c")
        base = wid * n_per_w
        pltpu.sync_copy(idx_hbm.at[pl.ds(base, n_per_w)], idx_v)

        def chunk(ci, carry):
            off = base + ci * CH
            cx = pltpu.async_copy(x_hbm.at[pl.ds(off, CH)], x_v, sx)
            cr = pltpu.async_copy(tab_hbm.at[idx_v.at[pl.ds(ci * CH, CH)]], r_v, sr)
            cx.wait()
            cr.wait()

            def row(i, c2):
                for j in range(C // _L):
                    sl = pl.ds(j * _L, _L)
                    plsc.addupdate(x_v.at[i, sl], r_v[i, sl])
                return c2

            lax.fori_loop(0, CH, row, 0)
            pltpu.async_copy(x_v, out_hbm.at[pl.ds(off, CH)], so).wait()
            return carry

        lax.fori_loop(0, n_ch, chunk, 0)

    return k(x_nc, idx_n, table)


def kernel(x_btc, times_bt, embeddings_tc, offset):
    B, T, C = x_btc.shape
    x = x_btc.reshape(B * T, C)
    idx = times_bt.reshape(B * T).astype(jnp.int32)
    out = _sc_gather_add(x, idx, embeddings_tc)
    return out.reshape(B, T, C)


# speedup vs baseline: 2.5627x; 2.5627x over previous
import functools

import jax
import jax.numpy as jnp
from jax import lax
from jax.experimental import pallas as pl
from jax.experimental.pallas import tpu as pltpu
from jax.experimental.pallas import tpu_sc as plsc

_NC, _NS, _L = 2, 16, 16
_NW = _NC * _NS
_CH = 8
_NBUF = 4
_DEPTH = 2


def _sc_gather_add(x_nc, idx_n, table):
    N, C = x_nc.shape
    n_per_w = N // _NW
    CH, NBUF, D = _CH, _NBUF, _DEPTH
    n_ch = n_per_w // CH
    n_grp = n_ch // NBUF
    mesh = plsc.VectorSubcoreMesh(core_axis_name="c", subcore_axis_name="s")

    scratch = [
        pltpu.VMEM((n_per_w,), jnp.int32),
        pltpu.VMEM((NBUF, CH, C), jnp.float32),
        pltpu.VMEM((NBUF, CH, C), jnp.float32),
    ] + [pltpu.SemaphoreType.DMA] * (3 * NBUF)

    @functools.partial(
        pl.kernel,
        out_type=jax.ShapeDtypeStruct((N, C), jnp.float32),
        mesh=mesh,
        scratch_types=scratch,
    )
    def k(x_hbm, idx_hbm, tab_hbm, out_hbm, idx_v, xb, rb, *sems):
        sx = sems[:NBUF]
        sr = sems[NBUF:2 * NBUF]
        so = sems[2 * NBUF:]
        wid = lax.axis_index("s") * _NC + lax.axis_index("c")
        base = wid * n_per_w
        pltpu.sync_copy(idx_hbm.at[pl.ds(base, n_per_w)], idx_v)

        def issue_in(ci, b):
            off = base + ci * CH
            pltpu.async_copy(x_hbm.at[pl.ds(off, CH)], xb.at[b], sx[b])
            pltpu.async_copy(tab_hbm.at[idx_v.at[pl.ds(ci * CH, CH)]], rb.at[b], sr[b])

        def wait_in(ci, b):
            off = base + ci * CH
            pltpu.make_async_copy(x_hbm.at[pl.ds(off, CH)], xb.at[b], sx[b]).wait()
            pltpu.make_async_copy(
                tab_hbm.at[idx_v.at[pl.ds(ci * CH, CH)]], rb.at[b], sr[b]).wait()

        def issue_out(ci, b):
            off = base + ci * CH
            pltpu.async_copy(xb.at[b], out_hbm.at[pl.ds(off, CH)], so[b])

        def wait_out(ci, b):
            off = base + ci * CH
            pltpu.make_async_copy(xb.at[b], out_hbm.at[pl.ds(off, CH)], so[b]).wait()

        def add_rows(b):
            def row(i, c2):
                for j in range(C // _L):
                    sl = pl.ds(j * _L, _L)
                    plsc.addupdate(xb.at[b, i, sl], rb[b, i, sl])
                return c2

            lax.fori_loop(0, CH, row, 0)

        for p in range(D):
            issue_in(p, p)

        def group(g, carry):
            for b in range(NBUF):
                ci = g * NBUF + b
                wait_in(ci, b)
                nci = ci + D
                nb = (b + D) % NBUF

                @pl.when(nci < n_ch)
                def _():
                    @pl.when(nci >= NBUF)
                    def _():
                        wait_out(nci - NBUF, nb)

                    issue_in(nci, nb)

                add_rows(b)
                issue_out(ci, b)
            return carry

        lax.fori_loop(0, n_grp, group, 0)
        for t in range(NBUF):
            ci = n_ch - NBUF + t
            wait_out(ci, ci % NBUF)

    return k(x_nc, idx_n, table)


def kernel(x_btc, times_bt, embeddings_tc, offset):
    B, T, C = x_btc.shape
    x = x_btc.reshape(B * T, C)
    idx = times_bt.reshape(B * T).astype(jnp.int32)
    out = _sc_gather_add(x, idx, embeddings_tc)
    return out.reshape(B, T, C)
